# baseline (speedup 1.0000x reference)
import jax
import jax.numpy as jnp
from jax.experimental import pallas as pl
from jax.experimental.pallas import tpu as pltpu


def _body(row_ref, col_ref, x_ref, o_ref, pos_ref):
    b = pl.program_id(0)

    @pl.when(b == 0)
    def _():
        row_t = row_ref[...].T
        col_t = col_ref[...].T
        pos_ref[...] = row_t[:, :, None] + col_t[:, None, :]

    o_ref[...] = x_ref[...] + pos_ref[...][None]


def kernel(x, row_embed, col_embed):
    B, D, H, W = x.shape
    return pl.pallas_call(
        _body,
        grid=(B,),
        in_specs=[
            pl.BlockSpec((H, D), lambda b: (0, 0)),
            pl.BlockSpec((W, D), lambda b: (0, 0)),
            pl.BlockSpec((1, D, H, W), lambda b: (b, 0, 0, 0)),
        ],
        out_specs=pl.BlockSpec((1, D, H, W), lambda b: (b, 0, 0, 0)),
        out_shape=jax.ShapeDtypeStruct(x.shape, x.dtype),
        scratch_shapes=[pltpu.VMEM((D, H, W), jnp.float32)],
    )(row_embed, col_embed, x)


# speedup vs baseline: 1.8271x; 1.8271x over previous
import jax
import jax.numpy as jnp
from jax.experimental import pallas as pl
from jax.experimental.pallas import tpu as pltpu


def _body(row_ref, col_ref, x_ref, o_ref, pos_ref):
    b = pl.program_id(0)

    @pl.when(b == 0)
    def _():
        row_t = row_ref[...].T
        col_t = col_ref[...].T
        pos3 = row_t[:, :, None] + col_t[:, None, :]
        pos_ref[...] = pos3.reshape(pos_ref.shape)

    o_ref[...] = x_ref[...] + pos_ref[...][None]


def kernel(x, row_embed, col_embed):
    B, D, H, W = x.shape
    xf = x.reshape(B, D, H * W)
    out = pl.pallas_call(
        _body,
        grid=(B,),
        in_specs=[
            pl.BlockSpec((H, D), lambda b: (0, 0)),
            pl.BlockSpec((W, D), lambda b: (0, 0)),
            pl.BlockSpec((1, D, H * W), lambda b: (b, 0, 0)),
        ],
        out_specs=pl.BlockSpec((1, D, H * W), lambda b: (b, 0, 0)),
        out_shape=jax.ShapeDtypeStruct(xf.shape, x.dtype),
        scratch_shapes=[pltpu.VMEM((D, H * W), jnp.float32)],
    )(row_embed, col_embed, xf)
    return out.reshape(B, D, H, W)
